# baseline (speedup 1.0000x reference)
import functools

import jax
import jax.numpy as jnp
from jax import lax
from jax.experimental import pallas as pl
from jax.experimental.pallas import tpu as pltpu
from jax.experimental.pallas import tpu_sc as plsc

VOCAB = 100000
EMBED_DIM = 64
BATCH = 4096
SEQ_LEN = 200

NC = 2
NS = 16
NW = NC * NS

B_PER_W = BATCH // NW
SPLIT = 96
K = 2
GROUPS = B_PER_W // K


@functools.partial(
    pl.kernel,
    out_type=jax.ShapeDtypeStruct((BATCH, SEQ_LEN, EMBED_DIM), jnp.float32),
    mesh=plsc.VectorSubcoreMesh(core_axis_name="c", subcore_axis_name="s"),
    compiler_params=pltpu.CompilerParams(use_tc_tiling_on_sc=False),
    scratch_types=[
        pltpu.VMEM((B_PER_W, SEQ_LEN), jnp.int32),
        pltpu.VMEM((2, K, SEQ_LEN, EMBED_DIM), jnp.float32),
        pltpu.SemaphoreType.DMA,
        pltpu.SemaphoreType.DMA,
        pltpu.SemaphoreType.DMA,
    ],
)
def _embed_lookup(idx_hbm, table_hbm, out_hbm, idx_v, rows_v, gsem,
                  ssem0, ssem1):
    ssem = (ssem0, ssem1)
    wid = lax.axis_index("s") * NC + lax.axis_index("c")
    b0 = wid * B_PER_W
    pltpu.sync_copy(idx_hbm.at[pl.ds(b0, B_PER_W)], idx_v)

    def halves(g, b):
        bb = g * K + b
        pieces = []
        for s0, n in ((0, SPLIT), (SPLIT, SEQ_LEN - SPLIT)):
            idx = idx_v.at[bb, pl.ds(s0, n)]
            pieces.append((idx, pl.ds(s0, n)))
        return bb, pieces

    def gather_fire(g, h):
        for b in range(K):
            _, pieces = halves(g, b)
            for idx, dst in pieces:
                pltpu.async_copy(table_hbm.at[idx], rows_v.at[h, b, dst],
                                 gsem)

    def gather_drain(g, h):
        for b in range(K):
            _, pieces = halves(g, b)
            for idx, dst in pieces:
                pltpu.make_async_copy(table_hbm.at[idx],
                                      rows_v.at[h, b, dst], gsem).wait()

    def store_fire(g, h):
        for b in range(K):
            bb, _ = halves(g, b)
            pltpu.async_copy(rows_v.at[h, b], out_hbm.at[b0 + bb], ssem[h])

    def store_drain(g, h):
        for b in range(K):
            bb, _ = halves(g, b)
            pltpu.make_async_copy(rows_v.at[h, b], out_hbm.at[b0 + bb],
                                  ssem[h]).wait()

    for h in range(2):
        gather_fire(h, h)
        gather_drain(h, h)
        store_fire(h, h)

    def group_pair(p, carry):
        for h in range(2):
            g = 2 * p + h
            store_drain(g - 2, h)
            gather_fire(g, h)
            gather_drain(g, h)
            store_fire(g, h)
        return carry

    lax.fori_loop(1, GROUPS // 2, group_pair, 0)

    store_drain(GROUPS - 2, 0)
    store_drain(GROUPS - 1, 1)


def kernel(x, word2vec_matrix):
    return _embed_lookup(x, word2vec_matrix)
